# baseline (speedup 1.0000x reference)
import functools

import jax
import jax.numpy as jnp
import numpy as np
from jax import lax
from jax.experimental import pallas as pl
from jax.experimental.pallas import tpu as pltpu
from jax.experimental.pallas import tpu_sc as plsc

_RADIUS = 0.12
_NSAMPLE = 32
_B, _N, _NP, _C = 4, 8192, 2048, 64
_NCHUNK = _N // 32
_QB = 256
_NW = 32
_QPW = (_B * _NP) // _NW
_WPB = _NW // _B
_FC = 4


def _pack_weights():
    j = np.arange(_N)
    g = j // 32
    k = j % 32
    wlo = np.zeros((_N, _NCHUNK), np.float32)
    whi = np.zeros((_N, _NCHUNK), np.float32)
    lo = k < 16
    wlo[j[lo], g[lo]] = (2.0 ** k[lo])
    whi[j[~lo], g[~lo]] = (2.0 ** (k[~lo] - 16))
    return wlo, whi


_WLO_NP, _WHI_NP = _pack_weights()


def _mask_words_body(nq_ref, xyzt_ref, wlo_ref, whi_ref, out_ref):
    q = nq_ref[0]
    p = xyzt_ref[0]
    qx, qy, qz = q[:, 0:1], q[:, 1:2], q[:, 2:3]
    px, py, pz = p[0:1, :], p[1:2, :], p[2:3, :]
    qq = qx * qx + qy * qy + qz * qz
    pp = px * px + py * py + pz * pz
    qp = jnp.dot(q.astype(jnp.bfloat16), p.astype(jnp.bfloat16),
                 preferred_element_type=jnp.float32)
    d2 = qq + pp - 2.0 * qp
    maskf = (d2 < (_RADIUS * _RADIUS)).astype(jnp.bfloat16)
    lo = jnp.dot(maskf, wlo_ref[...], preferred_element_type=jnp.float32)
    hi = jnp.dot(maskf, whi_ref[...], preferred_element_type=jnp.float32)
    out_ref[0] = lo.astype(jnp.int32) | (hi.astype(jnp.int32) << 16)


def _mask_words(new_xyz, xyz_t, wlo, whi, *, interpret=False):
    return pl.pallas_call(
        _mask_words_body,
        grid=(_B, _NP // _QB),
        in_specs=[
            pl.BlockSpec((1, _QB, 3), lambda b, i: (b, i, 0)),
            pl.BlockSpec((1, 3, _N), lambda b, i: (b, 0, 0)),
            pl.BlockSpec((_N, _NCHUNK), lambda b, i: (0, 0)),
            pl.BlockSpec((_N, _NCHUNK), lambda b, i: (0, 0)),
        ],
        out_specs=pl.BlockSpec((1, _QB, _NCHUNK), lambda b, i: (b, i, 0)),
        out_shape=jax.ShapeDtypeStruct((_B, _NP, _NCHUNK), jnp.int32),
        interpret=interpret,
    )(new_xyz, xyz_t, wlo, whi)


def _sc_body(words_hbm, xyzt_hbm, nqt_hbm, feat_hbm,
             cnt_hbm, out_hbm,
             xrow, yrow, zrow, f0, f1, f2, f3,
             idxflat, stag0, stag1, stag2, stag3,
             wflat, wordlist, qxrow, qyrow, qzrow, cntbuf, candbuf):
    cid = lax.axis_index("c")
    sid = lax.axis_index("s")
    wid = sid * 2 + cid
    b = wid // _WPB
    q0 = (wid % _WPB) * _QPW
    iota = lax.iota(jnp.int32, 16)
    one = jnp.int32(1)
    zero = jnp.int32(0)

    pltpu.sync_copy(xyzt_hbm.at[b, 0], xrow)
    pltpu.sync_copy(xyzt_hbm.at[b, 1], yrow)
    pltpu.sync_copy(xyzt_hbm.at[b, 2], zrow)
    pltpu.sync_copy(nqt_hbm.at[b, 0, pl.ds(q0, _QPW)], qxrow)
    pltpu.sync_copy(nqt_hbm.at[b, 1, pl.ds(q0, _QPW)], qyrow)
    pltpu.sync_copy(nqt_hbm.at[b, 2, pl.ds(q0, _QPW)], qzrow)

    def per_qchunk(qc, carry):
        pltpu.sync_copy(
            words_hbm.at[b, pl.ds((q0 + qc * 64) * _NCHUNK, 64 * _NCHUNK)],
            wflat)

        def per_q(qi, carry2):
            wbase = qi * _NCHUNK
            running = jnp.zeros((16,), jnp.int32)
            for v in range(16):
                wv = wflat[pl.ds(wbase + v * 16, 16)]
                nz = wv != 0
                cpos = plsc.cumsum(jnp.where(nz, one, zero))
                pos = running + cpos - 1
                plsc.store_scatter(wordlist, [pos], iota + v * 16,
                                   mask=nz & (pos < _NSAMPLE))
                running = running + plsc.all_reduce_population_count(nz)
            nwords = jnp.minimum(running, _NSAMPLE)

            def sel_body(k, f):
                wid_v = plsc.load_gather(wordlist, [jnp.full((16,), k, jnp.int32)])
                wid_v = wid_v & (_NCHUNK - 1)
                word = plsc.load_gather(wflat, [wbase + wid_v])
                act = (k < nwords) & (f < _NSAMPLE)
                base = wid_v * 32
                m0 = (((word >> iota) & one) == one) & act
                m1 = (((word >> (iota + 16)) & one) == one) & act
                c0 = plsc.cumsum(jnp.where(m0, one, zero))
                n0 = plsc.all_reduce_population_count(m0)
                plsc.store_scatter(candbuf, [c0 + (f - 1)],
                                   iota + base, mask=m0)
                c1 = plsc.cumsum(jnp.where(m1, one, zero))
                n1 = plsc.all_reduce_population_count(m1)
                plsc.store_scatter(candbuf, [c1 + (f + n0 - 1)],
                                   iota + base + 16, mask=m1)
                return f + n0 + n1

            found = lax.fori_loop(0, _NSAMPLE, sel_body,
                                  jnp.zeros((16,), jnp.int32))
            cnt_q = jnp.minimum(found, _NSAMPLE)
            qabs = qc * 64 + qi
            cv0 = candbuf[pl.ds(0, 16)]
            cv1 = candbuf[pl.ds(16, 16)]
            first = cv0[0]
            v0 = jnp.where(iota < cnt_q, cv0, first)
            v1 = jnp.where(iota + 16 < cnt_q, cv1, first)
            v0 = jnp.where(cnt_q > 0, v0, zero)
            v1 = jnp.where(cnt_q > 0, v1, zero)
            idxflat[pl.ds(qabs * _NSAMPLE, 16)] = v0
            idxflat[pl.ds(qabs * _NSAMPLE + 16, 16)] = v1
            plsc.store_scatter(cntbuf, [jnp.full((16,), qabs, jnp.int32)],
                               cnt_q, mask=iota == 0)
            return carry2

        return lax.fori_loop(0, 64, per_q, carry)

    lax.fori_loop(0, _QPW // 64, per_qchunk, jnp.int32(0))
    pltpu.sync_copy(cntbuf, cnt_hbm.at[b, pl.ds(q0, _QPW)])

    stags = (stag0, stag1, stag2, stag3)

    def per_qg_xyz(qg, carry):
        qxv = qxrow[pl.ds(qg * 16, 16)]
        qyv = qyrow[pl.ds(qg * 16, 16)]
        qzv = qzrow[pl.ds(qg * 16, 16)]
        for j in range(16):
            qi = qg * 16 + j
            qiv = jnp.full((16,), qi, jnp.int32)
            for h in range(2):
                idxv = idxflat[pl.ds(qi * _NSAMPLE + h * 16, 16)]
                gx = plsc.load_gather(xrow, [idxv])
                gy = plsc.load_gather(yrow, [idxv])
                gz = plsc.load_gather(zrow, [idxv])
                rows = iota + h * 16
                plsc.store_scatter(stag0, [rows, qiv], gx - qxv[j])
                plsc.store_scatter(stag1, [rows, qiv], gy - qyv[j])
                plsc.store_scatter(stag2, [rows, qiv], gz - qzv[j])
        return carry

    lax.fori_loop(0, _QPW // 16, per_qg_xyz, jnp.int32(0))
    for r in range(3):
        pltpu.sync_copy(stags[r], out_hbm.at[b, r, :, pl.ds(q0, _QPW)])

    frows = (f0, f1, f2, f3)

    def per_cc(cc, carry):
        for r in range(_FC):
            pltpu.sync_copy(feat_hbm.at[b, cc * _FC + r], frows[r])

        def per_q_feat(qi, c2):
            qiv = jnp.full((16,), qi, jnp.int32)
            for h in range(2):
                idxv = idxflat[pl.ds(qi * _NSAMPLE + h * 16, 16)]
                rows = iota + h * 16
                for r in range(_FC):
                    g = plsc.load_gather(frows[r], [idxv])
                    plsc.store_scatter(stags[r], [rows, qiv], g)
            return c2

        lax.fori_loop(0, _QPW, per_q_feat, jnp.int32(0))
        for r in range(_FC):
            pltpu.sync_copy(stags[r],
                            out_hbm.at[b, 3 + cc * _FC + r, :, pl.ds(q0, _QPW)])
        return carry

    lax.fori_loop(0, _C // _FC, per_cc, jnp.int32(0))


def _sc_call(words, xyz_t, new_xyz_t, features):
    mesh = plsc.VectorSubcoreMesh(core_axis_name="c", subcore_axis_name="s",
                                  num_cores=2, num_subcores=16)
    kern = pl.kernel(
        _sc_body,
        out_type=(
            jax.ShapeDtypeStruct((_B, _NP), jnp.int32),
            jax.ShapeDtypeStruct((_B, 3 + _C, _NSAMPLE, _NP), jnp.float32),
        ),
        mesh=mesh,
        compiler_params=pltpu.CompilerParams(use_tc_tiling_on_sc=False,
                                            needs_layout_passes=False),
        scratch_types=[
            pltpu.VMEM((_N,), jnp.float32),
            pltpu.VMEM((_N,), jnp.float32),
            pltpu.VMEM((_N,), jnp.float32),
            pltpu.VMEM((_N,), jnp.float32),
            pltpu.VMEM((_N,), jnp.float32),
            pltpu.VMEM((_N,), jnp.float32),
            pltpu.VMEM((_N,), jnp.float32),
            pltpu.VMEM((_QPW * _NSAMPLE,), jnp.int32),
            pltpu.VMEM((_NSAMPLE, _QPW), jnp.float32),
            pltpu.VMEM((_NSAMPLE, _QPW), jnp.float32),
            pltpu.VMEM((_NSAMPLE, _QPW), jnp.float32),
            pltpu.VMEM((_NSAMPLE, _QPW), jnp.float32),
            pltpu.VMEM((64 * _NCHUNK,), jnp.int32),
            pltpu.VMEM((_NSAMPLE,), jnp.int32),
            pltpu.VMEM((_QPW,), jnp.float32),
            pltpu.VMEM((_QPW,), jnp.float32),
            pltpu.VMEM((_QPW,), jnp.float32),
            pltpu.VMEM((_QPW,), jnp.int32),
            pltpu.VMEM((64,), jnp.int32),
        ],
    )
    return kern(words, xyz_t, new_xyz_t, features)


@jax.jit
def kernel(xyz, new_xyz, features):
    xyz_t = jnp.transpose(xyz, (0, 2, 1))
    new_xyz_t = jnp.transpose(new_xyz, (0, 2, 1))
    wlo = jnp.asarray(_WLO_NP, jnp.bfloat16)
    whi = jnp.asarray(_WHI_NP, jnp.bfloat16)
    words = _mask_words(new_xyz, xyz_t, wlo, whi)
    words = words.reshape(_B, _NP * _NCHUNK)
    cnt, out_t = _sc_call(words, xyz_t, new_xyz_t, features)
    return cnt, jnp.transpose(out_t, (0, 1, 3, 2))


# speedup vs baseline: 1.0482x; 1.0482x over previous
import functools

import jax
import jax.numpy as jnp
import numpy as np
from jax import lax
from jax.experimental import pallas as pl
from jax.experimental.pallas import tpu as pltpu
from jax.experimental.pallas import tpu_sc as plsc

_RADIUS = 0.12
_NSAMPLE = 32
_B, _N, _NP, _C = 4, 8192, 2048, 64
_NCHUNK = _N // 32
_QB = 256
_NW = 32
_QPW = (_B * _NP) // _NW
_WPB = _NW // _B
_FC = 4


def _pack_weights():
    j = np.arange(_N)
    g = j // 32
    k = j % 32
    wlo = np.zeros((_N, _NCHUNK), np.float32)
    whi = np.zeros((_N, _NCHUNK), np.float32)
    lo = k < 16
    wlo[j[lo], g[lo]] = (2.0 ** k[lo])
    whi[j[~lo], g[~lo]] = (2.0 ** (k[~lo] - 16))
    return wlo, whi


_WLO_NP, _WHI_NP = _pack_weights()


def _mask_words_body(nq_ref, xyzt_ref, wlo_ref, whi_ref, out_ref):
    q = nq_ref[0]
    p = xyzt_ref[0]
    qx, qy, qz = q[:, 0:1], q[:, 1:2], q[:, 2:3]
    px, py, pz = p[0:1, :], p[1:2, :], p[2:3, :]
    qq = qx * qx + qy * qy + qz * qz
    pp = px * px + py * py + pz * pz
    qp = jnp.dot(q.astype(jnp.bfloat16), p.astype(jnp.bfloat16),
                 preferred_element_type=jnp.float32)
    d2 = qq + pp - 2.0 * qp
    maskf = (d2 < (_RADIUS * _RADIUS)).astype(jnp.bfloat16)
    lo = jnp.dot(maskf, wlo_ref[...], preferred_element_type=jnp.float32)
    hi = jnp.dot(maskf, whi_ref[...], preferred_element_type=jnp.float32)
    out_ref[0] = lo.astype(jnp.int32) | (hi.astype(jnp.int32) << 16)


def _mask_words(new_xyz, xyz_t, wlo, whi, *, interpret=False):
    return pl.pallas_call(
        _mask_words_body,
        grid=(_B, _NP // _QB),
        in_specs=[
            pl.BlockSpec((1, _QB, 3), lambda b, i: (b, i, 0)),
            pl.BlockSpec((1, 3, _N), lambda b, i: (b, 0, 0)),
            pl.BlockSpec((_N, _NCHUNK), lambda b, i: (0, 0)),
            pl.BlockSpec((_N, _NCHUNK), lambda b, i: (0, 0)),
        ],
        out_specs=pl.BlockSpec((1, _QB, _NCHUNK), lambda b, i: (b, i, 0)),
        out_shape=jax.ShapeDtypeStruct((_B, _NP, _NCHUNK), jnp.int32),
        interpret=interpret,
    )(new_xyz, xyz_t, wlo, whi)


def _sc_body(words_hbm, xyzt_hbm, nqt_hbm, feat_hbm,
             cnt_hbm, out_hbm,
             xrow, yrow, zrow, f0, f1, f2, f3,
             idxflat, stag0, stag1, stag2, stag3,
             wflat, wordlist, qxrow, qyrow, qzrow, cntbuf, candbuf):
    cid = lax.axis_index("c")
    sid = lax.axis_index("s")
    wid = sid * 2 + cid
    b = wid // _WPB
    q0 = (wid % _WPB) * _QPW
    iota = lax.iota(jnp.int32, 16)
    one = jnp.int32(1)
    zero = jnp.int32(0)

    pltpu.sync_copy(xyzt_hbm.at[b, 0], xrow)
    pltpu.sync_copy(xyzt_hbm.at[b, 1], yrow)
    pltpu.sync_copy(xyzt_hbm.at[b, 2], zrow)
    pltpu.sync_copy(nqt_hbm.at[b, 0, pl.ds(q0, _QPW)], qxrow)
    pltpu.sync_copy(nqt_hbm.at[b, 1, pl.ds(q0, _QPW)], qyrow)
    pltpu.sync_copy(nqt_hbm.at[b, 2, pl.ds(q0, _QPW)], qzrow)

    def per_qchunk(qc, carry):
        pltpu.sync_copy(
            words_hbm.at[b, pl.ds((q0 + qc * 64) * _NCHUNK, 64 * _NCHUNK)],
            wflat)

        def per_q(qi, carry2):
            wbase = qi * _NCHUNK
            running = jnp.zeros((16,), jnp.int32)
            for v in range(16):
                wv = wflat[pl.ds(wbase + v * 16, 16)]
                nz = wv != 0
                cpos = plsc.cumsum(jnp.where(nz, one, zero))
                pos = running + cpos - 1
                plsc.store_scatter(wordlist, [pos], iota + v * 16,
                                   mask=nz & (pos < _NSAMPLE))
                running = running + plsc.all_reduce_population_count(nz)
            nwords = jnp.minimum(running, _NSAMPLE)

            def sel_body(k, f):
                wid_v = plsc.load_gather(wordlist, [jnp.full((16,), k, jnp.int32)])
                wid_v = wid_v & (_NCHUNK - 1)
                word = plsc.load_gather(wflat, [wbase + wid_v])
                act = (k < nwords) & (f < _NSAMPLE)
                base = wid_v * 32
                m0 = (((word >> iota) & one) == one) & act
                m1 = (((word >> (iota + 16)) & one) == one) & act
                c0 = plsc.cumsum(jnp.where(m0, one, zero))
                n0 = plsc.all_reduce_population_count(m0)
                plsc.store_scatter(candbuf, [c0 + (f - 1)],
                                   iota + base, mask=m0)
                c1 = plsc.cumsum(jnp.where(m1, one, zero))
                n1 = plsc.all_reduce_population_count(m1)
                plsc.store_scatter(candbuf, [c1 + (f + n0 - 1)],
                                   iota + base + 16, mask=m1)
                return f + n0 + n1

            found = lax.fori_loop(0, _NSAMPLE, sel_body,
                                  jnp.zeros((16,), jnp.int32))
            cnt_q = jnp.minimum(found, _NSAMPLE)
            qabs = qc * 64 + qi
            cv0 = candbuf[pl.ds(0, 16)]
            cv1 = candbuf[pl.ds(16, 16)]
            first = cv0[0]
            v0 = jnp.where(iota < cnt_q, cv0, first)
            v1 = jnp.where(iota + 16 < cnt_q, cv1, first)
            v0 = jnp.where(cnt_q > 0, v0, zero)
            v1 = jnp.where(cnt_q > 0, v1, zero)
            idxflat[pl.ds(qabs * _NSAMPLE, 16)] = v0
            idxflat[pl.ds(qabs * _NSAMPLE + 16, 16)] = v1
            plsc.store_scatter(cntbuf, [jnp.full((16,), qabs, jnp.int32)],
                               cnt_q, mask=iota == 0)
            return carry2

        return lax.fori_loop(0, 64, per_q, carry)

    lax.fori_loop(0, _QPW // 64, per_qchunk, jnp.int32(0))
    pltpu.sync_copy(cntbuf, cnt_hbm.at[b, pl.ds(q0, _QPW)])

    stags = (stag0, stag1, stag2, stag3)

    def per_qg_xyz(qg, carry):
        qxv = qxrow[pl.ds(qg * 16, 16)]
        qyv = qyrow[pl.ds(qg * 16, 16)]
        qzv = qzrow[pl.ds(qg * 16, 16)]
        for j in range(16):
            qi = qg * 16 + j
            for h in range(2):
                idxv = idxflat[pl.ds(qi * _NSAMPLE + h * 16, 16)]
                gx = plsc.load_gather(xrow, [idxv])
                gy = plsc.load_gather(yrow, [idxv])
                gz = plsc.load_gather(zrow, [idxv])
                stag0[qi, pl.ds(h * 16, 16)] = gx - qxv[j]
                stag1[qi, pl.ds(h * 16, 16)] = gy - qyv[j]
                stag2[qi, pl.ds(h * 16, 16)] = gz - qzv[j]
        return carry

    lax.fori_loop(0, _QPW // 16, per_qg_xyz, jnp.int32(0))
    for r in range(3):
        pltpu.sync_copy(stags[r], out_hbm.at[b, r, pl.ds(q0, _QPW)])

    frows = (f0, f1, f2, f3)

    def per_cc(cc, carry):
        for r in range(_FC):
            pltpu.sync_copy(feat_hbm.at[b, cc * _FC + r], frows[r])

        def per_q_feat(qi, c2):
            for h in range(2):
                idxv = idxflat[pl.ds(qi * _NSAMPLE + h * 16, 16)]
                for r in range(_FC):
                    g = plsc.load_gather(frows[r], [idxv])
                    stags[r][qi, pl.ds(h * 16, 16)] = g
            return c2

        lax.fori_loop(0, _QPW, per_q_feat, jnp.int32(0))
        for r in range(_FC):
            pltpu.sync_copy(stags[r],
                            out_hbm.at[b, 3 + cc * _FC + r, pl.ds(q0, _QPW)])
        return carry

    lax.fori_loop(0, _C // _FC, per_cc, jnp.int32(0))


def _sc_call(words, xyz_t, new_xyz_t, features):
    mesh = plsc.VectorSubcoreMesh(core_axis_name="c", subcore_axis_name="s",
                                  num_cores=2, num_subcores=16)
    kern = pl.kernel(
        _sc_body,
        out_type=(
            jax.ShapeDtypeStruct((_B, _NP), jnp.int32),
            jax.ShapeDtypeStruct((_B, 3 + _C, _NP, _NSAMPLE), jnp.float32),
        ),
        mesh=mesh,
        compiler_params=pltpu.CompilerParams(use_tc_tiling_on_sc=False,
                                            needs_layout_passes=False),
        scratch_types=[
            pltpu.VMEM((_N,), jnp.float32),
            pltpu.VMEM((_N,), jnp.float32),
            pltpu.VMEM((_N,), jnp.float32),
            pltpu.VMEM((_N,), jnp.float32),
            pltpu.VMEM((_N,), jnp.float32),
            pltpu.VMEM((_N,), jnp.float32),
            pltpu.VMEM((_N,), jnp.float32),
            pltpu.VMEM((_QPW * _NSAMPLE,), jnp.int32),
            pltpu.VMEM((_QPW, _NSAMPLE), jnp.float32),
            pltpu.VMEM((_QPW, _NSAMPLE), jnp.float32),
            pltpu.VMEM((_QPW, _NSAMPLE), jnp.float32),
            pltpu.VMEM((_QPW, _NSAMPLE), jnp.float32),
            pltpu.VMEM((64 * _NCHUNK,), jnp.int32),
            pltpu.VMEM((_NSAMPLE,), jnp.int32),
            pltpu.VMEM((_QPW,), jnp.float32),
            pltpu.VMEM((_QPW,), jnp.float32),
            pltpu.VMEM((_QPW,), jnp.float32),
            pltpu.VMEM((_QPW,), jnp.int32),
            pltpu.VMEM((64,), jnp.int32),
        ],
    )
    return kern(words, xyz_t, new_xyz_t, features)


@jax.jit
def kernel(xyz, new_xyz, features):
    xyz_t = jnp.transpose(xyz, (0, 2, 1))
    new_xyz_t = jnp.transpose(new_xyz, (0, 2, 1))
    wlo = jnp.asarray(_WLO_NP, jnp.bfloat16)
    whi = jnp.asarray(_WHI_NP, jnp.bfloat16)
    words = _mask_words(new_xyz, xyz_t, wlo, whi)
    words = words.reshape(_B, _NP * _NCHUNK)
    cnt, out = _sc_call(words, xyz_t, new_xyz_t, features)
    return cnt, out


# speedup vs baseline: 1.2041x; 1.1487x over previous
import functools

import jax
import jax.numpy as jnp
import numpy as np
from jax import lax
from jax.experimental import pallas as pl
from jax.experimental.pallas import tpu as pltpu
from jax.experimental.pallas import tpu_sc as plsc

_RADIUS = 0.12
_NSAMPLE = 32
_B, _N, _NP, _C = 4, 8192, 2048, 64
_NCHUNK = _N // 32
_QB = 256
_NW = 32
_QPW = (_B * _NP) // _NW
_WPB = _NW // _B
_FC = 4


def _pack_weights():
    j = np.arange(_N)
    g = j // 32
    k = j % 32
    wlo = np.zeros((_N, _NCHUNK), np.float32)
    whi = np.zeros((_N, _NCHUNK), np.float32)
    lo = k < 16
    wlo[j[lo], g[lo]] = (2.0 ** k[lo])
    whi[j[~lo], g[~lo]] = (2.0 ** (k[~lo] - 16))
    return wlo, whi


_WLO_NP, _WHI_NP = _pack_weights()


def _mask_words_body(nq_ref, xyzt_ref, wlo_ref, whi_ref, out_ref):
    q = nq_ref[0]
    p = xyzt_ref[0]
    qx, qy, qz = q[:, 0:1], q[:, 1:2], q[:, 2:3]
    px, py, pz = p[0:1, :], p[1:2, :], p[2:3, :]
    qq = qx * qx + qy * qy + qz * qz
    pp = px * px + py * py + pz * pz
    def _rbf(v):
        return v.astype(jnp.bfloat16).astype(jnp.float32)
    qp = (_rbf(qx) * _rbf(px) + _rbf(qy) * _rbf(py)) + _rbf(qz) * _rbf(pz)
    d2 = qq + pp - 2.0 * qp
    maskf = (d2 < (_RADIUS * _RADIUS)).astype(jnp.bfloat16)
    lo = jnp.dot(maskf, wlo_ref[...], preferred_element_type=jnp.float32)
    hi = jnp.dot(maskf, whi_ref[...], preferred_element_type=jnp.float32)
    out_ref[0] = lo.astype(jnp.int32) | (hi.astype(jnp.int32) << 16)


def _mask_words(new_xyz, xyz_t, wlo, whi, *, interpret=False):
    return pl.pallas_call(
        _mask_words_body,
        grid=(_B, _NP // _QB),
        in_specs=[
            pl.BlockSpec((1, _QB, 3), lambda b, i: (b, i, 0)),
            pl.BlockSpec((1, 3, _N), lambda b, i: (b, 0, 0)),
            pl.BlockSpec((_N, _NCHUNK), lambda b, i: (0, 0)),
            pl.BlockSpec((_N, _NCHUNK), lambda b, i: (0, 0)),
        ],
        out_specs=pl.BlockSpec((1, _QB, _NCHUNK), lambda b, i: (b, i, 0)),
        out_shape=jax.ShapeDtypeStruct((_B, _NP, _NCHUNK), jnp.int32),
        interpret=interpret,
    )(new_xyz, xyz_t, wlo, whi)


def _sc_body(words_hbm, xyzt_hbm, nqt_hbm, feat_hbm,
             cnt_hbm, out_hbm,
             xrow, yrow, zrow, f0, f1, f2, f3,
             idxflat, stag0, stag1, stag2, stag3,
             wflat, wlbuf, nwbuf, qxrow, qyrow, qzrow, cntbuf):
    cid = lax.axis_index("c")
    sid = lax.axis_index("s")
    wid = sid * 2 + cid
    b = wid // _WPB
    q0 = (wid % _WPB) * _QPW
    iota = lax.iota(jnp.int32, 16)
    one = jnp.int32(1)
    zero = jnp.int32(0)

    pltpu.sync_copy(xyzt_hbm.at[b, 0], xrow)
    pltpu.sync_copy(xyzt_hbm.at[b, 1], yrow)
    pltpu.sync_copy(xyzt_hbm.at[b, 2], zrow)
    pltpu.sync_copy(nqt_hbm.at[b, 0, pl.ds(q0, _QPW)], qxrow)
    pltpu.sync_copy(nqt_hbm.at[b, 1, pl.ds(q0, _QPW)], qyrow)
    pltpu.sync_copy(nqt_hbm.at[b, 2, pl.ds(q0, _QPW)], qzrow)

    def per_qchunk(qc, carry):
        pltpu.sync_copy(
            words_hbm.at[b, pl.ds((q0 + qc * 64) * _NCHUNK, 64 * _NCHUNK)],
            wflat)

        def scan_q(qi, carry2):
            wbase = qi * _NCHUNK
            qabs = qc * 64 + qi
            running = jnp.zeros((16,), jnp.int32)
            for v in range(16):
                wv = wflat[pl.ds(wbase + v * 16, 16)]
                nz = wv != 0
                cpos = plsc.cumsum(jnp.where(nz, one, zero))
                pos = running + cpos - 1
                plsc.store_scatter(wlbuf, [qabs * _NSAMPLE + pos],
                                   iota + v * 16,
                                   mask=nz & (pos < _NSAMPLE))
                running = running + plsc.all_reduce_population_count(nz)
            plsc.store_scatter(nwbuf, [jnp.full((16,), qabs, jnp.int32)],
                               jnp.minimum(running, _NSAMPLE),
                               mask=iota == 0)
            return carry2

        lax.fori_loop(0, 64, scan_q, carry)

        for g in range(4):
            qloc = g * 16 + iota
            qabsv = qc * 64 + qloc
            nw_v = nwbuf[pl.ds(qc * 64 + g * 16, 16)]

            def sel_step(k, f):
                wid = plsc.load_gather(wlbuf, [qabsv * _NSAMPLE + k])
                wid = wid & (_NCHUNK - 1)
                word = plsc.load_gather(wflat, [qloc * _NCHUNK + wid])
                act = k < nw_v
                cand0 = wid * 32
                for bit in range(32):
                    m = (((word >> bit) & one) == one) & act & (f < _NSAMPLE)
                    plsc.store_scatter(idxflat, [qabsv * _NSAMPLE + f],
                                       cand0 + bit, mask=m)
                    f = f + jnp.where(m, one, zero)
                return f

            f_fin = lax.fori_loop(0, _NSAMPLE, sel_step,
                                  jnp.zeros((16,), jnp.int32))
            cntbuf[pl.ds(qc * 64 + g * 16, 16)] = f_fin
        return carry

    lax.fori_loop(0, _QPW // 64, per_qchunk, jnp.int32(0))

    def pad_q(qi, carry):
        cntv = plsc.load_gather(cntbuf, [jnp.full((16,), qi, jnp.int32)])
        firstv = plsc.load_gather(idxflat,
                                  [jnp.full((16,), qi * _NSAMPLE, jnp.int32)])
        for h in range(2):
            sl = iota + h * 16
            v = idxflat[pl.ds(qi * _NSAMPLE + h * 16, 16)]
            v = jnp.where(sl < cntv, v, firstv)
            v = jnp.where(cntv > 0, v, zero)
            idxflat[pl.ds(qi * _NSAMPLE + h * 16, 16)] = v
        return carry

    lax.fori_loop(0, _QPW, pad_q, jnp.int32(0))
    pltpu.sync_copy(cntbuf, cnt_hbm.at[b, pl.ds(q0, _QPW)])

    stags = (stag0, stag1, stag2, stag3)

    def per_qg_xyz(qg, carry):
        qxv = qxrow[pl.ds(qg * 16, 16)]
        qyv = qyrow[pl.ds(qg * 16, 16)]
        qzv = qzrow[pl.ds(qg * 16, 16)]
        for j in range(16):
            qi = qg * 16 + j
            for h in range(2):
                idxv = idxflat[pl.ds(qi * _NSAMPLE + h * 16, 16)]
                gx = plsc.load_gather(xrow, [idxv])
                gy = plsc.load_gather(yrow, [idxv])
                gz = plsc.load_gather(zrow, [idxv])
                stag0[qi, pl.ds(h * 16, 16)] = gx - qxv[j]
                stag1[qi, pl.ds(h * 16, 16)] = gy - qyv[j]
                stag2[qi, pl.ds(h * 16, 16)] = gz - qzv[j]
        return carry

    lax.fori_loop(0, _QPW // 16, per_qg_xyz, jnp.int32(0))
    for r in range(3):
        pltpu.sync_copy(stags[r], out_hbm.at[b, r, pl.ds(q0, _QPW)])

    frows = (f0, f1, f2, f3)

    def per_cc(cc, carry):
        for r in range(_FC):
            pltpu.sync_copy(feat_hbm.at[b, cc * _FC + r], frows[r])

        def per_q_feat(qi, c2):
            for h in range(2):
                idxv = idxflat[pl.ds(qi * _NSAMPLE + h * 16, 16)]
                for r in range(_FC):
                    g = plsc.load_gather(frows[r], [idxv])
                    stags[r][qi, pl.ds(h * 16, 16)] = g
            return c2

        lax.fori_loop(0, _QPW, per_q_feat, jnp.int32(0))
        for r in range(_FC):
            pltpu.sync_copy(stags[r],
                            out_hbm.at[b, 3 + cc * _FC + r, pl.ds(q0, _QPW)])
        return carry

    lax.fori_loop(0, _C // _FC, per_cc, jnp.int32(0))


def _sc_call(words, xyz_t, new_xyz_t, features):
    mesh = plsc.VectorSubcoreMesh(core_axis_name="c", subcore_axis_name="s",
                                  num_cores=2, num_subcores=16)
    kern = pl.kernel(
        _sc_body,
        out_type=(
            jax.ShapeDtypeStruct((_B, _NP), jnp.int32),
            jax.ShapeDtypeStruct((_B, 3 + _C, _NP, _NSAMPLE), jnp.float32),
        ),
        mesh=mesh,
        compiler_params=pltpu.CompilerParams(use_tc_tiling_on_sc=False,
                                            needs_layout_passes=False),
        scratch_types=[
            pltpu.VMEM((_N,), jnp.float32),
            pltpu.VMEM((_N,), jnp.float32),
            pltpu.VMEM((_N,), jnp.float32),
            pltpu.VMEM((_N,), jnp.float32),
            pltpu.VMEM((_N,), jnp.float32),
            pltpu.VMEM((_N,), jnp.float32),
            pltpu.VMEM((_N,), jnp.float32),
            pltpu.VMEM((_QPW * _NSAMPLE,), jnp.int32),
            pltpu.VMEM((_QPW, _NSAMPLE), jnp.float32),
            pltpu.VMEM((_QPW, _NSAMPLE), jnp.float32),
            pltpu.VMEM((_QPW, _NSAMPLE), jnp.float32),
            pltpu.VMEM((_QPW, _NSAMPLE), jnp.float32),
            pltpu.VMEM((64 * _NCHUNK,), jnp.int32),
            pltpu.VMEM((_QPW * _NSAMPLE,), jnp.int32),
            pltpu.VMEM((_QPW,), jnp.int32),
            pltpu.VMEM((_QPW,), jnp.float32),
            pltpu.VMEM((_QPW,), jnp.float32),
            pltpu.VMEM((_QPW,), jnp.float32),
            pltpu.VMEM((_QPW,), jnp.int32),
        ],
    )
    return kern(words, xyz_t, new_xyz_t, features)


@jax.jit
def kernel(xyz, new_xyz, features):
    xyz_t = jnp.transpose(xyz, (0, 2, 1))
    new_xyz_t = jnp.transpose(new_xyz, (0, 2, 1))
    wlo = jnp.asarray(_WLO_NP, jnp.bfloat16)
    whi = jnp.asarray(_WHI_NP, jnp.bfloat16)
    words = _mask_words(new_xyz, xyz_t, wlo, whi)
    words = words.reshape(_B, _NP * _NCHUNK)
    cnt, out = _sc_call(words, xyz_t, new_xyz_t, features)
    return cnt, out
